# baseline (speedup 1.0000x reference)
import functools

import jax
import jax.numpy as jnp
from jax.experimental import pallas as pl
from jax.experimental.pallas import tpu as pltpu


def _fused_body(x_ref, we_ref, be_ref, wc_ref, bc_ref, y_ref, s_ref, *, keep):
    j = pl.program_id(1)
    xb = x_ref[0]
    hT = jnp.dot(xb, we_ref[...], preferred_element_type=jnp.float32)
    hT = hT + be_ref[...]

    bits = jax.lax.bitcast_convert_type(hT, jnp.uint32)
    neg = bits >= jnp.uint32(0x80000000)
    key = jnp.where(neg, ~bits, bits | jnp.uint32(0x80000000))

    th = bits.shape[1]

    def to_s16(u32vals):
        u16 = (u32vals ^ jnp.uint32(0x8000)).astype(jnp.uint16)
        return jax.lax.bitcast_convert_type(u16, jnp.int16)

    hi = to_s16(key >> jnp.uint32(16))
    lo = to_s16(key & jnp.uint32(0xFFFF))

    def bit32(i):
        return jnp.left_shift(jnp.uint32(1), (15 - i).astype(jnp.uint32))

    oneb = jnp.bfloat16(1)
    zerob = jnp.bfloat16(0)
    onesrow = jnp.ones((1, hT.shape[0]), jnp.bfloat16)

    def count_slabbed(arr, mask_fn):
        mk = jnp.where(mask_fn(arr), oneb, zerob)
        cnt = jax.lax.dot_general(onesrow, mk, (((1,), (0,)), ((), ())),
                                  preferred_element_type=jnp.float32)
        return cnt.astype(jnp.int32)

    def cnt16(arr, cand):
        c = to_s16(cand)
        return count_slabbed(arr, lambda a: a >= c)

    def p1(i, t):
        cand = t | bit32(i)
        return jnp.where(cnt16(hi, cand) >= keep, cand, t)

    t16 = jax.lax.fori_loop(0, 16, p1, jnp.zeros((1, th), jnp.uint32),
                            unroll=4)
    t16s = to_s16(t16)

    c_gt = count_slabbed(hi, lambda a: a > t16s)
    c_eq = count_slabbed(hi, lambda a: a == t16s)
    act = jnp.where(hi == t16s, lo, jnp.int16(-32768))

    def p2_cond(c):
        i, _, cc = c
        return (i < 16) & jnp.logical_not(jnp.all(cc == keep))

    def p2_body(c):
        i, t, cc = c
        for u in range(4):
            cand = t | bit32(i + u)
            cnt = c_gt + cnt16(act, cand)
            acc = cnt >= keep
            t = jnp.where(acc, cand, t)
            cc = jnp.where(acc, cnt, cc)
        return (i + 4, t, cc)

    _, tlo, _ = jax.lax.while_loop(
        p2_cond, p2_body,
        (jnp.int32(0), jnp.zeros((1, th), jnp.uint32), c_gt + c_eq))

    t = (t16 << jnp.uint32(16)) | tlo
    tbits = jnp.where(t >= jnp.uint32(0x80000000),
                      t & jnp.uint32(0x7FFFFFFF), ~t)
    thr = jax.lax.bitcast_convert_type(tbits, jnp.float32)
    spT = jnp.where(hT >= thr, hT, 0.0)
    sp = spT.T
    s_ref[0] = sp
    yj = jnp.dot(wc_ref[...], sp, preferred_element_type=jnp.float32)

    @pl.when(j == 0)
    def _init():
        y_ref[0] = yj + bc_ref[0][:, None]

    @pl.when(j != 0)
    def _acc():
        y_ref[0] = y_ref[0] + yj


def _run(x, W_expand, b_expand, W_contract, b_contract, keep, th):
    B, D, L = x.shape
    H = W_expand.shape[0]
    nj = H // th
    be2 = b_expand.reshape(1, H)
    bc2 = b_contract.reshape(1, D)
    xT16 = x.transpose(0, 2, 1).astype(jnp.bfloat16)
    weT16 = W_expand.T.astype(jnp.bfloat16)
    grid = (B, nj)
    y, sparse = pl.pallas_call(
        functools.partial(_fused_body, keep=keep),
        grid=grid,
        in_specs=[
            pl.BlockSpec((1, L, D), lambda b, j: (b, 0, 0)),
            pl.BlockSpec((D, th), lambda b, j: (0, j)),
            pl.BlockSpec((1, th), lambda b, j: (0, j)),
            pl.BlockSpec((D, th), lambda b, j: (0, j)),
            pl.BlockSpec((1, D), lambda b, j: (0, 0)),
        ],
        out_specs=[
            pl.BlockSpec((1, D, L), lambda b, j: (b, 0, 0)),
            pl.BlockSpec((1, th, L), lambda b, j: (b, j, 0)),
        ],
        out_shape=[
            jax.ShapeDtypeStruct((B, D, L), jnp.float32),
            jax.ShapeDtypeStruct((B, H, L), jnp.float32),
        ],
        compiler_params=pltpu.CompilerParams(
            dimension_semantics=("parallel", "arbitrary"),
        ),
    )(xT16, weT16, be2, W_contract, bc2)
    return y, sparse


def kernel(x, W_expand, b_expand, W_contract, b_contract):
    return _run(x, W_expand, b_expand, W_contract, b_contract, keep=64, th=512)


# speedup vs baseline: 1.3415x; 1.3415x over previous
import functools

import jax
import jax.numpy as jnp
from jax.experimental import pallas as pl
from jax.experimental.pallas import tpu as pltpu


def _fused_body(x_ref, we_ref, be_ref, wc_ref, bc_ref, y_ref, s_ref, *, keep):
    j = pl.program_id(1)
    xb = x_ref[0]
    hT = jnp.dot(xb, we_ref[...], preferred_element_type=jnp.float32)
    hT = hT + be_ref[...]

    bits = jax.lax.bitcast_convert_type(hT, jnp.uint32)
    neg = bits >= jnp.uint32(0x80000000)
    key = jnp.where(neg, ~bits, bits | jnp.uint32(0x80000000))

    th = bits.shape[1]

    def to_s16(u32vals):
        u16 = (u32vals ^ jnp.uint32(0x8000)).astype(jnp.uint16)
        return jax.lax.bitcast_convert_type(u16, jnp.int16)

    hi = to_s16(key >> jnp.uint32(16))
    lo = to_s16(key & jnp.uint32(0xFFFF))

    def bit32(i):
        return jnp.left_shift(jnp.uint32(1), (15 - i).astype(jnp.uint32))

    one16 = jnp.int16(1)
    zero16 = jnp.int16(0)

    def count_slabbed(arr, mask_fn):
        slab = 16
        accs = [None] * 4
        for k in range(arr.shape[0] // slab):
            mk = jnp.where(mask_fn(arr[k * slab:(k + 1) * slab]),
                           one16, zero16)
            i = k % 4
            accs[i] = mk if accs[i] is None else accs[i] + mk
        acc = (accs[0] + accs[1]) + (accs[2] + accs[3])
        return jnp.sum(acc.astype(jnp.int32), axis=0, keepdims=True)

    def cnt16(arr, cand):
        c = to_s16(cand)
        return count_slabbed(arr, lambda a: a >= c)

    def p1(i, t):
        cand = t | bit32(i)
        return jnp.where(cnt16(hi, cand) >= keep, cand, t)

    t16 = jax.lax.fori_loop(0, 16, p1, jnp.zeros((1, th), jnp.uint32),
                            unroll=4)
    t16s = to_s16(t16)

    c_gt = count_slabbed(hi, lambda a: a > t16s)
    c_eq = count_slabbed(hi, lambda a: a == t16s)
    act = jnp.where(hi == t16s, lo, jnp.int16(-32768))

    def p2_cond(c):
        i, _, cc = c
        return (i < 16) & jnp.logical_not(jnp.all(cc == keep))

    def p2_body(c):
        i, t, cc = c
        for u in range(4):
            cand = t | bit32(i + u)
            cnt = c_gt + cnt16(act, cand)
            acc = cnt >= keep
            t = jnp.where(acc, cand, t)
            cc = jnp.where(acc, cnt, cc)
        return (i + 4, t, cc)

    _, tlo, _ = jax.lax.while_loop(
        p2_cond, p2_body,
        (jnp.int32(0), jnp.zeros((1, th), jnp.uint32), c_gt + c_eq))

    t = (t16 << jnp.uint32(16)) | tlo
    tbits = jnp.where(t >= jnp.uint32(0x80000000),
                      t & jnp.uint32(0x7FFFFFFF), ~t)
    thr = jax.lax.bitcast_convert_type(tbits, jnp.float32)
    spT = jnp.where(hT >= thr, hT, 0.0)
    sp = spT.T
    s_ref[0] = sp
    yj = jnp.dot(wc_ref[...], sp, preferred_element_type=jnp.float32)

    @pl.when(j == 0)
    def _init():
        y_ref[0] = yj + bc_ref[0][:, None]

    @pl.when(j != 0)
    def _acc():
        y_ref[0] = y_ref[0] + yj


def _run(x, W_expand, b_expand, W_contract, b_contract, keep, th):
    B, D, L = x.shape
    H = W_expand.shape[0]
    nj = H // th
    be2 = b_expand.reshape(1, H)
    bc2 = b_contract.reshape(1, D)
    xT16 = x.transpose(0, 2, 1).astype(jnp.bfloat16)
    weT16 = W_expand.T.astype(jnp.bfloat16)
    grid = (B, nj)
    y, sparse = pl.pallas_call(
        functools.partial(_fused_body, keep=keep),
        grid=grid,
        in_specs=[
            pl.BlockSpec((1, L, D), lambda b, j: (b, 0, 0)),
            pl.BlockSpec((D, th), lambda b, j: (0, j)),
            pl.BlockSpec((1, th), lambda b, j: (0, j)),
            pl.BlockSpec((D, th), lambda b, j: (0, j)),
            pl.BlockSpec((1, D), lambda b, j: (0, 0)),
        ],
        out_specs=[
            pl.BlockSpec((1, D, L), lambda b, j: (b, 0, 0)),
            pl.BlockSpec((1, th, L), lambda b, j: (b, j, 0)),
        ],
        out_shape=[
            jax.ShapeDtypeStruct((B, D, L), jnp.float32),
            jax.ShapeDtypeStruct((B, H, L), jnp.float32),
        ],
        compiler_params=pltpu.CompilerParams(
            dimension_semantics=("parallel", "arbitrary"),
        ),
    )(xT16, weT16, be2, W_contract, bc2)
    return y, sparse


def kernel(x, W_expand, b_expand, W_contract, b_contract):
    return _run(x, W_expand, b_expand, W_contract, b_contract, keep=64, th=512)
